# Initial kernel scaffold; baseline (speedup 1.0000x reference)
import jax
import jax.numpy as jnp
from jax.experimental import pallas as pl


def kernel(patch, patch_lib):
    raise NotImplementedError("write your pallas kernel here")


# speedup vs baseline: 3.6565x; 3.6565x over previous
import functools

import jax
import jax.numpy as jnp
from jax.experimental import pallas as pl
from jax.experimental.pallas import tpu as pltpu

Q = 256
K = 512
BN = 2048
BIG_I = 2**30


def _pass1_body(nblocks, nrows, patch_ref, lib_ref, sstar_ref, jstar_ref,
                mtest_ref, minsq_ref, argidx_ref):
    i = pl.program_id(0)
    patch = patch_ref[...]
    block = lib_ref[...]
    b2col = jax.lax.dot_general(
        block * block, jnp.ones((K, 1), jnp.float32),
        (((1,), (0,)), ((), ())),
        preferred_element_type=jnp.float32)
    block_aug = jnp.concatenate([block, b2col], axis=1)
    patch_aug = jnp.concatenate(
        [-2.0 * patch, jnp.ones((Q, 1), jnp.float32)], axis=1)
    scores = jax.lax.dot_general(
        patch_aug, block_aug, (((1,), (1,)), ((), ())),
        preferred_element_type=jnp.float32)
    cols = jax.lax.broadcasted_iota(jnp.int32, (1, BN), 1)
    valid = (cols + i * BN) < nrows
    scores = jnp.where(valid, scores, jnp.float32(jnp.inf))
    bmin = jnp.min(scores, axis=1).reshape(Q, 1)
    barg = (jnp.argmin(scores, axis=1).astype(jnp.int32)
            .reshape(Q, 1) + i * BN)

    @pl.when(i == 0)
    def _():
        minsq_ref[...] = bmin
        argidx_ref[...] = barg

    @pl.when(i > 0)
    def _():
        prev = minsq_ref[...]
        take = bmin < prev
        minsq_ref[...] = jnp.where(take, bmin, prev)
        argidx_ref[...] = jnp.where(take, barg, argidx_ref[...])

    @pl.when(i == nblocks - 1)
    def _():
        a2 = jnp.sum(patch * patch, axis=1).reshape(Q, 1)
        minval = jnp.sqrt(jnp.maximum(a2 + minsq_ref[...], 1e-12))
        sstar = jnp.max(minval)
        rows = jax.lax.broadcasted_iota(jnp.int32, (Q, 1), 0)
        s_idx = jnp.min(jnp.where(minval == sstar, rows, BIG_I))
        rowsel = rows == s_idx
        sstar_ref[...] = sstar.reshape(1, 1)
        jstar_ref[...] = jnp.sum(jnp.where(rowsel, argidx_ref[...], 0)).reshape(1, 1)
        mtest_ref[...] = jnp.sum(
            jnp.where(rowsel, patch, 0.0), axis=0, keepdims=True)


def _pass2_body(nblocks, nrows, js_ref, mstar_blk_ref, mtest_ref, sstar_ref,
                lib_ref, out_ref, vals_s, idx_s, tv_s):
    i = pl.program_id(0)
    jstar = js_ref[0]

    @pl.when(i == 0)
    def _():
        for k in range(3):
            vals_s[k] = jnp.float32(jnp.inf)
            idx_s[k] = jnp.int32(BIG_I + k)
            tv_s[k] = jnp.float32(0.0)

    r8 = jax.lax.broadcasted_iota(jnp.int32, (8, 1), 0)
    rsel = r8 == (jstar % 8)
    mstar = jnp.sum(jnp.where(rsel, mstar_blk_ref[...], 0.0), axis=0,
                    keepdims=True)
    mtest = mtest_ref[...]
    q = jnp.concatenate([mstar, mtest], axis=0)

    block = lib_ref[...]
    b2 = jnp.sum(block * block, axis=1)[None, :]
    prod = jax.lax.dot_general(
        q, block, (((1,), (1,)), ((), ())),
        preferred_element_type=jnp.float32)
    w = b2 - 2.0 * prod[0:1, :]
    a2_t = jnp.sum(mtest * mtest)
    t = a2_t + b2 - 2.0 * prod[1:2, :]

    cols = jax.lax.broadcasted_iota(jnp.int32, (1, BN), 1)
    w = jnp.where((cols + i * BN) < nrows, w, jnp.float32(jnp.inf))
    cands = []
    for _ in range(3):
        m = jnp.min(w)
        loc = jnp.min(jnp.where(w == m, cols, BIG_I))
        sel = cols == loc
        tval = jnp.sum(jnp.where(sel, t, 0.0))
        cands.append((m, loc + i * BN, tval))
        w = jnp.where(sel, jnp.float32(jnp.inf), w)

    for k in range(3):
        cands.append((vals_s[k], idx_s[k], tv_s[k]))

    def cswap(a, b):
        sw = (b[0] < a[0]) | ((b[0] == a[0]) & (b[1] < a[1]))
        lo = tuple(jnp.where(sw, y, x) for x, y in zip(a, b))
        hi = tuple(jnp.where(sw, x, y) for x, y in zip(a, b))
        return lo, hi

    c = cands
    for p in range(6):
        for qi in range(5 - p):
            c[qi], c[qi + 1] = cswap(c[qi], c[qi + 1])

    for k in range(3):
        vals_s[k] = c[k][0]
        idx_s[k] = c[k][1]
        tv_s[k] = c[k][2]

    @pl.when(i == nblocks - 1)
    def _():
        d = jnp.sqrt(jnp.float32(K))
        sstar = sstar_ref[0, 0]
        k1 = jnp.sqrt(jnp.maximum(tv_s[1], 0.0) + 1e-12)
        k2 = jnp.sqrt(jnp.maximum(tv_s[2], 0.0) + 1e-12)
        wgt = 1.0 - jnp.exp(sstar / d) / (jnp.exp(k1 / d) + jnp.exp(k2 / d))
        out_ref[...] = (wgt * sstar).reshape(1, 1)


@jax.jit
def kernel(patch, patch_lib):
    n = patch_lib.shape[0]
    nblocks = (n + BN - 1) // BN

    sstar, jstar, mtest = pl.pallas_call(
        functools.partial(_pass1_body, nblocks, n),
        grid=(nblocks,),
        in_specs=[
            pl.BlockSpec((Q, K), lambda i: (0, 0)),
            pl.BlockSpec((BN, K), lambda i: (i, 0)),
        ],
        out_specs=[
            pl.BlockSpec((1, 1), lambda i: (0, 0)),
            pl.BlockSpec((1, 1), lambda i: (0, 0)),
            pl.BlockSpec((1, K), lambda i: (0, 0)),
        ],
        out_shape=[
            jax.ShapeDtypeStruct((1, 1), jnp.float32),
            jax.ShapeDtypeStruct((1, 1), jnp.int32),
            jax.ShapeDtypeStruct((1, K), jnp.float32),
        ],
        scratch_shapes=[
            pltpu.VMEM((Q, 1), jnp.float32),
            pltpu.VMEM((Q, 1), jnp.int32),
        ],
    )(patch, patch_lib)

    s = pl.pallas_call(
        functools.partial(_pass2_body, nblocks, n),
        grid_spec=pltpu.PrefetchScalarGridSpec(
            num_scalar_prefetch=1,
            grid=(nblocks,),
            in_specs=[
                pl.BlockSpec((8, K), lambda i, js: (js[0] // 8, 0)),
                pl.BlockSpec((1, K), lambda i, js: (0, 0)),
                pl.BlockSpec((1, 1), lambda i, js: (0, 0)),
                pl.BlockSpec((BN, K), lambda i, js: (i, 0)),
            ],
            out_specs=pl.BlockSpec((1, 1), lambda i, js: (0, 0)),
            scratch_shapes=[
                pltpu.SMEM((3,), jnp.float32),
                pltpu.SMEM((3,), jnp.int32),
                pltpu.SMEM((3,), jnp.float32),
            ],
        ),
        out_shape=jax.ShapeDtypeStruct((1, 1), jnp.float32),
    )(jstar.reshape(1), patch_lib, mtest, sstar, patch_lib)

    return s[0, 0]


# speedup vs baseline: 4.8126x; 1.3162x over previous
import functools

import jax
import jax.numpy as jnp
from jax.experimental import pallas as pl
from jax.experimental.pallas import tpu as pltpu

Q = 256
K = 512
BN = 2048
BIG_I = 2**30


def _pass1_body(nblocks, nrows, patch_ref, lib_ref, sstar_ref, jstar_ref,
                mtest_ref, b2_ref, minsq_ref, argidx_ref):
    i = pl.program_id(0)
    patch = patch_ref[...]
    block = lib_ref[...]
    b2row = jax.lax.dot_general(
        jnp.ones((1, K), jnp.float32), block * block,
        (((1,), (1,)), ((), ())),
        preferred_element_type=jnp.float32)
    prod = jax.lax.dot_general(
        -2.0 * patch, block, (((1,), (1,)), ((), ())),
        preferred_element_type=jnp.float32)
    scores = prod + b2row
    cols = jax.lax.broadcasted_iota(jnp.int32, (1, BN), 1)
    valid = (cols + i * BN) < nrows
    scores = jnp.where(valid, scores, jnp.float32(jnp.inf))
    b2_ref[...] = b2row
    bmin = jnp.min(scores, axis=1).reshape(Q, 1)
    barg = (jnp.argmin(scores, axis=1).astype(jnp.int32)
            .reshape(Q, 1) + i * BN)

    @pl.when(i == 0)
    def _():
        minsq_ref[...] = bmin
        argidx_ref[...] = barg

    @pl.when(i > 0)
    def _():
        prev = minsq_ref[...]
        take = bmin < prev
        minsq_ref[...] = jnp.where(take, bmin, prev)
        argidx_ref[...] = jnp.where(take, barg, argidx_ref[...])

    @pl.when(i == nblocks - 1)
    def _():
        a2 = jnp.sum(patch * patch, axis=1).reshape(Q, 1)
        minval = jnp.sqrt(jnp.maximum(a2 + minsq_ref[...], 1e-12))
        sstar = jnp.max(minval)
        rows = jax.lax.broadcasted_iota(jnp.int32, (Q, 1), 0)
        s_idx = jnp.min(jnp.where(minval == sstar, rows, BIG_I))
        rowsel = rows == s_idx
        sstar_ref[...] = sstar.reshape(1, 1)
        jstar_ref[...] = jnp.sum(
            jnp.where(rowsel, argidx_ref[...], 0)).reshape(1, 1)
        mtest_ref[...] = jnp.sum(
            jnp.where(rowsel, patch, 0.0), axis=0, keepdims=True)


def _pass2_body(nblocks, nrows, js_ref, mstar_blk_ref, mtest_ref, sstar_ref,
                lib_ref, b2_ref, out_ref, vals_s, idx_s, tv_s):
    i = pl.program_id(0)
    jstar = js_ref[0]

    @pl.when(i == 0)
    def _():
        for k in range(3):
            vals_s[k] = jnp.float32(jnp.inf)
            idx_s[k] = jnp.int32(BIG_I + k)
            tv_s[k] = jnp.float32(0.0)

    r8 = jax.lax.broadcasted_iota(jnp.int32, (8, 1), 0)
    rsel = r8 == (jstar % 8)
    mstar = jnp.sum(jnp.where(rsel, mstar_blk_ref[...], 0.0), axis=0,
                    keepdims=True)
    mtest = mtest_ref[...]
    q = jnp.concatenate([mstar, mtest], axis=0)

    block = lib_ref[...]
    b2row = b2_ref[...]
    prod = jax.lax.dot_general(
        -2.0 * q, block, (((1,), (1,)), ((), ())),
        preferred_element_type=jnp.float32)
    w = b2row + prod[0:1, :]
    a2_t = jnp.sum(mtest * mtest)
    t = a2_t + b2row + prod[1:2, :]

    cols = jax.lax.broadcasted_iota(jnp.int32, (1, BN), 1)
    w = jnp.where((cols + i * BN) < nrows, w, jnp.float32(jnp.inf))
    cands = []
    for _ in range(3):
        m = jnp.min(w)
        loc = jnp.min(jnp.where(w == m, cols, BIG_I))
        sel = cols == loc
        tval = jnp.sum(jnp.where(sel, t, 0.0))
        cands.append((m, loc + i * BN, tval))
        w = jnp.where(sel, jnp.float32(jnp.inf), w)

    for k in range(3):
        cands.append((vals_s[k], idx_s[k], tv_s[k]))

    def cswap(a, b):
        sw = (b[0] < a[0]) | ((b[0] == a[0]) & (b[1] < a[1]))
        lo = tuple(jnp.where(sw, y, x) for x, y in zip(a, b))
        hi = tuple(jnp.where(sw, x, y) for x, y in zip(a, b))
        return lo, hi

    c = cands
    for p in range(6):
        for qi in range(5 - p):
            c[qi], c[qi + 1] = cswap(c[qi], c[qi + 1])

    for k in range(3):
        vals_s[k] = c[k][0]
        idx_s[k] = c[k][1]
        tv_s[k] = c[k][2]

    @pl.when(i == nblocks - 1)
    def _():
        d = jnp.sqrt(jnp.float32(K))
        sstar = sstar_ref[0, 0]
        k1 = jnp.sqrt(jnp.maximum(tv_s[1], 0.0) + 1e-12)
        k2 = jnp.sqrt(jnp.maximum(tv_s[2], 0.0) + 1e-12)
        wgt = 1.0 - jnp.exp(sstar / d) / (jnp.exp(k1 / d) + jnp.exp(k2 / d))
        out_ref[...] = (wgt * sstar).reshape(1, 1)


@jax.jit
def kernel(patch, patch_lib):
    n = patch_lib.shape[0]
    nblocks = (n + BN - 1) // BN

    sstar, jstar, mtest, b2all = pl.pallas_call(
        functools.partial(_pass1_body, nblocks, n),
        grid=(nblocks,),
        in_specs=[
            pl.BlockSpec((Q, K), lambda i: (0, 0)),
            pl.BlockSpec((BN, K), lambda i: (i, 0)),
        ],
        out_specs=[
            pl.BlockSpec((1, 1), lambda i: (0, 0)),
            pl.BlockSpec((1, 1), lambda i: (0, 0)),
            pl.BlockSpec((1, K), lambda i: (0, 0)),
            pl.BlockSpec((1, BN), lambda i: (0, i)),
        ],
        out_shape=[
            jax.ShapeDtypeStruct((1, 1), jnp.float32),
            jax.ShapeDtypeStruct((1, 1), jnp.int32),
            jax.ShapeDtypeStruct((1, K), jnp.float32),
            jax.ShapeDtypeStruct((1, nblocks * BN), jnp.float32),
        ],
        scratch_shapes=[
            pltpu.VMEM((Q, 1), jnp.float32),
            pltpu.VMEM((Q, 1), jnp.int32),
        ],
    )(patch, patch_lib)

    s = pl.pallas_call(
        functools.partial(_pass2_body, nblocks, n),
        grid_spec=pltpu.PrefetchScalarGridSpec(
            num_scalar_prefetch=1,
            grid=(nblocks,),
            in_specs=[
                pl.BlockSpec((8, K), lambda i, js: (js[0] // 8, 0)),
                pl.BlockSpec((1, K), lambda i, js: (0, 0)),
                pl.BlockSpec((1, 1), lambda i, js: (0, 0)),
                pl.BlockSpec((BN, K), lambda i, js: (i, 0)),
                pl.BlockSpec((1, BN), lambda i, js: (0, i)),
            ],
            out_specs=pl.BlockSpec((1, 1), lambda i, js: (0, 0)),
            scratch_shapes=[
                pltpu.SMEM((3,), jnp.float32),
                pltpu.SMEM((3,), jnp.int32),
                pltpu.SMEM((3,), jnp.float32),
            ],
        ),
        out_shape=jax.ShapeDtypeStruct((1, 1), jnp.float32),
    )(jstar.reshape(1), patch_lib, mtest, sstar, patch_lib, b2all)

    return s[0, 0]


# speedup vs baseline: 4.8626x; 1.0104x over previous
import functools

import jax
import jax.numpy as jnp
from jax.experimental import pallas as pl
from jax.experimental.pallas import tpu as pltpu

Q = 256
K = 512
BN = 2048
BIG_I = 2**30


def _scores_block(patch, block):
    b2row = jax.lax.dot_general(
        jnp.ones((1, K), jnp.float32), block * block,
        (((1,), (1,)), ((), ())),
        preferred_element_type=jnp.float32)
    prod = jax.lax.dot_general(
        -2.0 * patch, block, (((1,), (1,)), ((), ())),
        preferred_element_type=jnp.float32)
    return prod + b2row, b2row


def _pass1_body(nblocks, nrows, patch_ref, lib_ref, sstar_ref, bstar_ref,
                mtest_ref, b2_ref, minsq_ref, blk_ref):
    i = pl.program_id(0)
    patch = patch_ref[...]
    block = lib_ref[...]
    scores, b2row = _scores_block(patch, block)
    b2_ref[...] = b2row

    def update(sc):
        bmin = jnp.min(sc, axis=1).reshape(Q, 1)
        prev = minsq_ref[...]
        take = bmin < prev
        minsq_ref[...] = jnp.where(take, bmin, prev)
        blk_ref[...] = jnp.where(take, i, blk_ref[...])

    @pl.when(i == 0)
    def _():
        minsq_ref[...] = jnp.full((Q, 1), jnp.inf, jnp.float32)
        blk_ref[...] = jnp.zeros((Q, 1), jnp.int32)

    @pl.when(i < nblocks - 1)
    def _():
        update(scores)

    @pl.when(i == nblocks - 1)
    def _():
        cols = jax.lax.broadcasted_iota(jnp.int32, (1, BN), 1)
        valid = (cols + i * BN) < nrows
        update(jnp.where(valid, scores, jnp.float32(jnp.inf)))

        a2 = jnp.sum(patch * patch, axis=1).reshape(Q, 1)
        minval = jnp.sqrt(jnp.maximum(a2 + minsq_ref[...], 1e-12))
        sstar = jnp.max(minval)
        rows = jax.lax.broadcasted_iota(jnp.int32, (Q, 1), 0)
        s_idx = jnp.min(jnp.where(minval == sstar, rows, BIG_I))
        rowsel = rows == s_idx
        sstar_ref[...] = sstar.reshape(1, 1)
        bstar_ref[...] = jnp.sum(
            jnp.where(rowsel, blk_ref[...], 0)).reshape(1, 1)
        mtest_ref[...] = jnp.sum(
            jnp.where(rowsel, patch, 0.0), axis=0, keepdims=True)


def _locate_body(nrows, bs_ref, mtest_ref, lib_ref, b2_ref, jstar_ref):
    bstar = bs_ref[0]
    mtest = mtest_ref[...]
    block = lib_ref[...]
    b2row = b2_ref[...]
    t = b2row + jax.lax.dot_general(
        -2.0 * mtest, block, (((1,), (1,)), ((), ())),
        preferred_element_type=jnp.float32)
    cols = jax.lax.broadcasted_iota(jnp.int32, (1, BN), 1)
    t = jnp.where((cols + bstar * BN) < nrows, t, jnp.float32(jnp.inf))
    m = jnp.min(t)
    loc = jnp.min(jnp.where(t == m, cols, BIG_I))
    jstar_ref[...] = (loc + bstar * BN).reshape(1, 1)


def _pass2_body(nblocks, nrows, js_ref, mstar_blk_ref, mtest_ref, sstar_ref,
                lib_ref, b2_ref, out_ref, vals_s, idx_s, tv_s):
    i = pl.program_id(0)
    jstar = js_ref[0]

    @pl.when(i == 0)
    def _():
        for k in range(3):
            vals_s[k] = jnp.float32(jnp.inf)
            idx_s[k] = jnp.int32(BIG_I + k)
            tv_s[k] = jnp.float32(0.0)

    r8 = jax.lax.broadcasted_iota(jnp.int32, (8, 1), 0)
    rsel = r8 == (jstar % 8)
    mstar = jnp.sum(jnp.where(rsel, mstar_blk_ref[...], 0.0), axis=0,
                    keepdims=True)
    mtest = mtest_ref[...]
    q = jnp.concatenate([mstar, mtest], axis=0)

    block = lib_ref[...]
    b2row = b2_ref[...]
    prod = jax.lax.dot_general(
        -2.0 * q, block, (((1,), (1,)), ((), ())),
        preferred_element_type=jnp.float32)
    w = b2row + prod[0:1, :]
    a2_t = jnp.sum(mtest * mtest)
    t = a2_t + b2row + prod[1:2, :]

    cols = jax.lax.broadcasted_iota(jnp.int32, (1, BN), 1)
    w = jnp.where((cols + i * BN) < nrows, w, jnp.float32(jnp.inf))
    cands = []
    for _ in range(3):
        m = jnp.min(w)
        loc = jnp.min(jnp.where(w == m, cols, BIG_I))
        sel = cols == loc
        tval = jnp.sum(jnp.where(sel, t, 0.0))
        cands.append((m, loc + i * BN, tval))
        w = jnp.where(sel, jnp.float32(jnp.inf), w)

    for k in range(3):
        cands.append((vals_s[k], idx_s[k], tv_s[k]))

    def cswap(a, b):
        sw = (b[0] < a[0]) | ((b[0] == a[0]) & (b[1] < a[1]))
        lo = tuple(jnp.where(sw, y, x) for x, y in zip(a, b))
        hi = tuple(jnp.where(sw, x, y) for x, y in zip(a, b))
        return lo, hi

    c = cands
    for p in range(6):
        for qi in range(5 - p):
            c[qi], c[qi + 1] = cswap(c[qi], c[qi + 1])

    for k in range(3):
        vals_s[k] = c[k][0]
        idx_s[k] = c[k][1]
        tv_s[k] = c[k][2]

    @pl.when(i == nblocks - 1)
    def _():
        d = jnp.sqrt(jnp.float32(K))
        sstar = sstar_ref[0, 0]
        k1 = jnp.sqrt(jnp.maximum(tv_s[1], 0.0) + 1e-12)
        k2 = jnp.sqrt(jnp.maximum(tv_s[2], 0.0) + 1e-12)
        wgt = 1.0 - jnp.exp(sstar / d) / (jnp.exp(k1 / d) + jnp.exp(k2 / d))
        out_ref[...] = (wgt * sstar).reshape(1, 1)


@jax.jit
def kernel(patch, patch_lib):
    n = patch_lib.shape[0]
    nblocks = (n + BN - 1) // BN

    sstar, bstar, mtest, b2all = pl.pallas_call(
        functools.partial(_pass1_body, nblocks, n),
        grid=(nblocks,),
        in_specs=[
            pl.BlockSpec((Q, K), lambda i: (0, 0)),
            pl.BlockSpec((BN, K), lambda i: (i, 0)),
        ],
        out_specs=[
            pl.BlockSpec((1, 1), lambda i: (0, 0)),
            pl.BlockSpec((1, 1), lambda i: (0, 0)),
            pl.BlockSpec((1, K), lambda i: (0, 0)),
            pl.BlockSpec((1, BN), lambda i: (0, i)),
        ],
        out_shape=[
            jax.ShapeDtypeStruct((1, 1), jnp.float32),
            jax.ShapeDtypeStruct((1, 1), jnp.int32),
            jax.ShapeDtypeStruct((1, K), jnp.float32),
            jax.ShapeDtypeStruct((1, nblocks * BN), jnp.float32),
        ],
        scratch_shapes=[
            pltpu.VMEM((Q, 1), jnp.float32),
            pltpu.VMEM((Q, 1), jnp.int32),
        ],
    )(patch, patch_lib)

    jstar = pl.pallas_call(
        functools.partial(_locate_body, n),
        grid_spec=pltpu.PrefetchScalarGridSpec(
            num_scalar_prefetch=1,
            grid=(1,),
            in_specs=[
                pl.BlockSpec((1, K), lambda i, bs: (0, 0)),
                pl.BlockSpec((BN, K), lambda i, bs: (bs[0], 0)),
                pl.BlockSpec((1, BN), lambda i, bs: (0, bs[0])),
            ],
            out_specs=pl.BlockSpec((1, 1), lambda i, bs: (0, 0)),
        ),
        out_shape=jax.ShapeDtypeStruct((1, 1), jnp.int32),
    )(bstar.reshape(1), mtest, patch_lib, b2all)

    s = pl.pallas_call(
        functools.partial(_pass2_body, nblocks, n),
        grid_spec=pltpu.PrefetchScalarGridSpec(
            num_scalar_prefetch=1,
            grid=(nblocks,),
            in_specs=[
                pl.BlockSpec((8, K), lambda i, js: (js[0] // 8, 0)),
                pl.BlockSpec((1, K), lambda i, js: (0, 0)),
                pl.BlockSpec((1, 1), lambda i, js: (0, 0)),
                pl.BlockSpec((BN, K), lambda i, js: (i, 0)),
                pl.BlockSpec((1, BN), lambda i, js: (0, i)),
            ],
            out_specs=pl.BlockSpec((1, 1), lambda i, js: (0, 0)),
            scratch_shapes=[
                pltpu.SMEM((3,), jnp.float32),
                pltpu.SMEM((3,), jnp.int32),
                pltpu.SMEM((3,), jnp.float32),
            ],
        ),
        out_shape=jax.ShapeDtypeStruct((1, 1), jnp.float32),
    )(jstar.reshape(1), patch_lib, mtest, sstar, patch_lib, b2all)

    return s[0, 0]
